# Initial kernel scaffold; baseline (speedup 1.0000x reference)
import jax
import jax.numpy as jnp
from jax.experimental import pallas as pl


def kernel(h, edge_index, edge_weight, Ws_w, Ws_b, Wn_w, Wn_b, Wu_w, Wu_b, lin_w, lin_b):
    raise NotImplementedError("write your pallas kernel here")


# speedup vs baseline: 3.1273x; 3.1273x over previous
import functools

import jax
import jax.numpy as jnp
from jax import lax
from jax.experimental import pallas as pl
from jax.experimental.pallas import tpu as pltpu
from jax.experimental.pallas import tpu_sc as plsc

N_NODES = 10000
N_EDGES = 320000
D = 128
DQ = 32
MW = 2 * DQ
NW = 32
CHUNK = 128
NCHUNKS = 79
EPT = NCHUNKS * CHUNK
E_PAD = EPT * NW
NACC = 10240
RPT = NACC // 16


def _sc_aggregate(h0, h1, h2, h3, srcb, dstb, wb):
    mesh = plsc.VectorSubcoreMesh(core_axis_name="c", subcore_axis_name="s")

    @functools.partial(
        pl.kernel,
        out_type=[
            jax.ShapeDtypeStruct((2, 4, NACC, MW), jnp.float32),
            jax.ShapeDtypeStruct((NW, NACC), jnp.float32),
        ],
        mesh=mesh,
        compiler_params=pltpu.CompilerParams(
            needs_layout_passes=False, use_tc_tiling_on_sc=False),
        scratch_types=[
            pltpu.VMEM((NCHUNKS, CHUNK), jnp.int32),
            pltpu.VMEM((NCHUNKS, CHUNK), jnp.int32),
            pltpu.VMEM((NCHUNKS, CHUNK), jnp.float32),
            pltpu.VMEM((CHUNK, DQ), jnp.float32),
            pltpu.VMEM((CHUNK, MW), jnp.float32),
            pltpu.VMEM((CHUNK, MW), jnp.float32),
            pltpu.VMEM((NACC,), jnp.float32),
            pltpu.VMEM_SHARED((NACC, MW), jnp.float32),
            pltpu.SemaphoreType.DMA,
        ],
    )
    def agg(h0_hbm, h1_hbm, h2_hbm, h3_hbm, src_hbm, dst_hbm, w_hbm,
            out_feat, out_deg,
            src_v, dst_v, w_v, rows_v, msg_v, zb_v, deg_v, acc, sem):
        hq_hbms = [h0_hbm, h1_hbm, h2_hbm, h3_hbm]
        cid = lax.axis_index("c")
        sid = lax.axis_index("s")
        wid = sid * 2 + cid

        pltpu.sync_copy(src_hbm.at[wid], src_v)
        pltpu.sync_copy(dst_hbm.at[wid], dst_v)
        pltpu.sync_copy(w_hbm.at[wid], w_v)

        zeros16 = jnp.zeros((16,), jnp.float32)
        ones16 = jnp.ones((16,), jnp.float32)
        lanes = lax.iota(jnp.int32, 16)

        def fill_zb(e, _):
            for c in range(MW // 16):
                zb_v[e, pl.ds(c * 16, 16)] = zeros16
            return 0
        lax.fori_loop(0, CHUNK, fill_zb, 0)

        def fill_deg0(r, _):
            deg_v[pl.ds(r * 16, 16)] = zeros16
            return 0
        lax.fori_loop(0, NACC // 16, fill_deg0, 0)

        for p in range(4):
            for k in range(RPT // CHUNK):
                r0 = sid * RPT + k * CHUNK
                pltpu.sync_copy(zb_v, acc.at[pl.ds(r0, CHUNK), :])
            plsc.subcore_barrier()

            def chunk_body(j, _, do_deg, h_hbm):
                pltpu.async_copy(h_hbm.at[src_v.at[j]], rows_v, sem).wait()

                def group_body(g, _):
                    wv = w_v[j, pl.ds(g * 16, 16)]
                    if do_deg:
                        dv = dst_v[j, pl.ds(g * 16, 16)]
                    for l in range(16):
                        e = g * 16 + l
                        ws = wv[l]
                        for c in range(DQ // 16):
                            r = rows_v[e, pl.ds(c * 16, 16)]
                            msg_v[e, pl.ds(c * 16, 16)] = r * ws
                            msg_v[e, pl.ds(DQ + c * 16, 16)] = r
                        if do_deg:
                            plsc.addupdate_scatter(
                                deg_v, [dv], ones16, mask=lanes == l)
                    return 0
                lax.fori_loop(0, CHUNK // 16, group_body, 0)

                pltpu.sync_copy(msg_v, acc.at[dst_v.at[j]], add=True)
                return 0
            lax.fori_loop(0, NCHUNKS,
                          functools.partial(chunk_body, do_deg=(p == 0),
                                            h_hbm=hq_hbms[p]), 0)

            if p == 0:
                pltpu.sync_copy(deg_v, out_deg.at[wid])

            plsc.subcore_barrier()
            for k in range(RPT // CHUNK):
                r0 = sid * RPT + k * CHUNK
                pltpu.sync_copy(acc.at[pl.ds(r0, CHUNK), :], msg_v)
                pltpu.sync_copy(msg_v, out_feat.at[cid, p, pl.ds(r0, CHUNK), :])
            plsc.subcore_barrier()

    return agg(h0, h1, h2, h3, srcb, dstb, wb)


def _tc_body(h_ref, p_ref, deg_ref, ws_ref, wn_ref, wu_ref, ball_ref,
             lin_ref, linb_ref, out_ref):
    f32 = jnp.float32
    h = h_ref[...]
    p = p_ref[...]
    deg = jnp.sum(deg_ref[...], axis=0)
    r = (1.0 / jnp.maximum(deg, 1.0))[:, None]

    def mm_t(x, w):
        return lax.dot_general(x, w, (((1,), (1,)), ((), ())),
                               preferred_element_type=f32)

    wn = wn_ref[...]
    wu = wu_ref[...]
    hp = jnp.zeros_like(h)
    hu = jnp.zeros_like(h)
    for q in range(4):
        s1q = p[0, q, :, 0:DQ] + p[1, q, :, 0:DQ]
        s0q = p[0, q, :, DQ:MW] + p[1, q, :, DQ:MW]
        hp = hp + mm_t(s1q, wn[:, q * DQ:(q + 1) * DQ])
        hu = hu + mm_t(s0q, wu[:, q * DQ:(q + 1) * DQ])
    ht = mm_t(h, ws_ref[...]) + (hp + hu) * r + ball_ref[...]
    out_ref[...] = mm_t(ht, lin_ref[...]) + linb_ref[...]


def kernel(h, edge_index, edge_weight, Ws_w, Ws_b, Wn_w, Wn_b, Wu_w, Wu_b,
           lin_w, lin_b):
    src = edge_index[0].astype(jnp.int32)
    dst = edge_index[1].astype(jnp.int32)
    w = edge_weight.reshape(-1).astype(jnp.float32)

    pad = E_PAD - N_EDGES
    src = jnp.pad(src, (0, pad)).reshape(NW, NCHUNKS, CHUNK)
    dst = jnp.pad(dst, (0, pad), constant_values=N_NODES).reshape(
        NW, NCHUNKS, CHUNK)
    w = jnp.pad(w, (0, pad)).reshape(NW, NCHUNKS, CHUNK)

    hqs = [h[:, q * DQ:(q + 1) * DQ] + 0.0 for q in range(4)]

    p_feat, p_deg = _sc_aggregate(*hqs, src, dst, w)

    b_all = (Ws_b + Wn_b + Wu_b).reshape(1, D)
    linb = lin_b.reshape(1, D)

    BLK = 1024
    grid = NACC // BLK
    out = pl.pallas_call(
        _tc_body,
        grid=(grid,),
        in_specs=[
            pl.BlockSpec((BLK, D), lambda i: (i, 0)),
            pl.BlockSpec((2, 4, BLK, MW), lambda i: (0, 0, i, 0)),
            pl.BlockSpec((NW, BLK), lambda i: (0, i)),
            pl.BlockSpec((D, D), lambda i: (0, 0)),
            pl.BlockSpec((D, D), lambda i: (0, 0)),
            pl.BlockSpec((D, D), lambda i: (0, 0)),
            pl.BlockSpec((1, D), lambda i: (0, 0)),
            pl.BlockSpec((D, D), lambda i: (0, 0)),
            pl.BlockSpec((1, D), lambda i: (0, 0)),
        ],
        out_specs=pl.BlockSpec((BLK, D), lambda i: (i, 0)),
        out_shape=jax.ShapeDtypeStruct((N_NODES, D), jnp.float32),
    )(h, p_feat, p_deg, Ws_w, Wn_w, Wu_w, b_all, lin_w, linb)
    return out
